# baseline (speedup 1.0000x reference)
import jax
import jax.numpy as jnp
from jax.experimental import pallas as pl
from jax.experimental.pallas import tpu as pltpu

NUM_TOKENS = 32768
MODEL_DIM = 1024
NUM_EXPERTS = 64
BLOCK_T = 512
NUM_BLOCKS = NUM_TOKENS // BLOCK_T


def _gate_body(x_ref, w_ref, idx_ref, loc_ref, gate_ref, laux_ref,
               me_acc, cnt_acc):
    i = pl.program_id(0)

    @pl.when(i == 0)
    def _init():
        me_acc[...] = jnp.zeros_like(me_acc)
        cnt_acc[...] = jnp.zeros_like(cnt_acc)

    x = x_ref[...]
    w = w_ref[...]
    logits = jax.lax.dot_general(
        x, w, (((1,), (1,)), ((), ())),
        preferred_element_type=jnp.float32)

    rowmax = jnp.max(logits, axis=1, keepdims=True)
    eidx_f = jax.lax.broadcasted_iota(
        jnp.int32, (BLOCK_T, NUM_EXPERTS), 1).astype(jnp.float32)
    is_max = logits == rowmax
    idx_f = jnp.min(jnp.where(is_max, eidx_f, float(NUM_EXPERTS)),
                    axis=1, keepdims=True)

    exps = jnp.exp(logits - rowmax)
    mask = (eidx_f == idx_f).astype(jnp.float32)

    r = jax.lax.broadcasted_iota(jnp.int32, (BLOCK_T, BLOCK_T), 0)
    c = jax.lax.broadcasted_iota(jnp.int32, (BLOCK_T, BLOCK_T), 1)
    ltri = (c < r).astype(jnp.bfloat16)
    csum = jax.lax.dot_general(
        ltri, mask.astype(jnp.bfloat16), (((1,), (0,)), ((), ())),
        preferred_element_type=jnp.float32)

    carry = cnt_acc[...]

    pair = jnp.concatenate([exps, (csum + carry) * mask], axis=1)
    ones_col = jnp.ones((NUM_EXPERTS, 1), jnp.float32)
    zeros_col = jnp.zeros((NUM_EXPERTS, 1), jnp.float32)
    sel = jnp.concatenate(
        [jnp.concatenate([ones_col, zeros_col], axis=1),
         jnp.concatenate([zeros_col, ones_col], axis=1)], axis=0)
    red = jax.lax.dot_general(
        pair, sel, (((1,), (0,)), ((), ())),
        preferred_element_type=jnp.float32)
    denom = red[:, 0:1]
    loc = red[:, 1:2]
    gate = 1.0 / denom

    me_part = jnp.sum(exps * gate, axis=0, keepdims=True)
    ce_part = jnp.sum(mask, axis=0, keepdims=True)

    idx_ref[...] = idx_f.astype(jnp.int32)
    loc_ref[...] = loc.astype(jnp.int32)
    gate_ref[...] = gate
    me_acc[...] += me_part
    cnt_acc[...] += ce_part

    @pl.when(i == NUM_BLOCKS - 1)
    def _fin():
        laux_ref[0, 0] = (jnp.sum(me_acc[...] * cnt_acc[...])
                          * (NUM_EXPERTS / (NUM_TOKENS * NUM_TOKENS)))


def kernel(input, W):
    num_tokens, num_experts = NUM_TOKENS, NUM_EXPERTS
    capacity = int((num_tokens + num_experts - 1) // num_experts)

    idx2, loc2, gate2, laux = pl.pallas_call(
        _gate_body,
        grid=(NUM_BLOCKS,),
        in_specs=[
            pl.BlockSpec((BLOCK_T, MODEL_DIM), lambda i: (i, 0)),
            pl.BlockSpec((NUM_EXPERTS, MODEL_DIM), lambda i: (0, 0)),
        ],
        out_specs=[
            pl.BlockSpec((BLOCK_T, 1), lambda i: (i, 0)),
            pl.BlockSpec((BLOCK_T, 1), lambda i: (i, 0)),
            pl.BlockSpec((BLOCK_T, 1), lambda i: (i, 0)),
            pl.BlockSpec(memory_space=pltpu.SMEM),
        ],
        out_shape=[
            jax.ShapeDtypeStruct((NUM_TOKENS, 1), jnp.int32),
            jax.ShapeDtypeStruct((NUM_TOKENS, 1), jnp.int32),
            jax.ShapeDtypeStruct((NUM_TOKENS, 1), jnp.float32),
            jax.ShapeDtypeStruct((1, 1), jnp.float32),
        ],
        scratch_shapes=[
            pltpu.VMEM((1, NUM_EXPERTS), jnp.float32),
            pltpu.VMEM((1, NUM_EXPERTS), jnp.float32),
        ],
    )(input, W)

    return (laux[0, 0], idx2[:, 0], capacity, loc2[:, 0], gate2[:, 0],
            num_experts)


# speedup vs baseline: 1.1467x; 1.1467x over previous
import jax
import jax.numpy as jnp
from jax.experimental import pallas as pl
from jax.experimental.pallas import tpu as pltpu

NUM_TOKENS = 32768
MODEL_DIM = 1024
NUM_EXPERTS = 64
BLOCK_T = 1024
NUM_BLOCKS = NUM_TOKENS // BLOCK_T


def _gate_body(x_ref, w_ref, idx_ref, loc_ref, gate_ref, laux_ref,
               me_acc, cnt_acc):
    i = pl.program_id(0)

    @pl.when(i == 0)
    def _init():
        me_acc[...] = jnp.zeros_like(me_acc)
        cnt_acc[...] = jnp.zeros_like(cnt_acc)

    x = x_ref[...]
    w = w_ref[...]
    logits = jax.lax.dot_general(
        x, w, (((1,), (1,)), ((), ())),
        preferred_element_type=jnp.float32)

    rowmax = jnp.max(logits, axis=1, keepdims=True)
    eidx_f = jax.lax.broadcasted_iota(
        jnp.int32, (BLOCK_T, NUM_EXPERTS), 1).astype(jnp.float32)
    is_max = logits == rowmax
    idx_f = jnp.min(jnp.where(is_max, eidx_f, float(NUM_EXPERTS)),
                    axis=1, keepdims=True)

    exps = jnp.exp(logits - rowmax)
    mask = (eidx_f == idx_f).astype(jnp.float32)

    r = jax.lax.broadcasted_iota(jnp.int32, (BLOCK_T, BLOCK_T), 0)
    c = jax.lax.broadcasted_iota(jnp.int32, (BLOCK_T, BLOCK_T), 1)
    ltri = (c < r).astype(jnp.bfloat16)
    csum = jax.lax.dot_general(
        ltri, mask.astype(jnp.bfloat16), (((1,), (0,)), ((), ())),
        preferred_element_type=jnp.float32)

    carry = cnt_acc[...]

    pair = jnp.concatenate([exps, (csum + carry) * mask], axis=1)
    ones_col = jnp.ones((NUM_EXPERTS, 1), jnp.float32)
    zeros_col = jnp.zeros((NUM_EXPERTS, 1), jnp.float32)
    sel = jnp.concatenate(
        [jnp.concatenate([ones_col, zeros_col], axis=1),
         jnp.concatenate([zeros_col, ones_col], axis=1)], axis=0)
    red = jax.lax.dot_general(
        pair, sel, (((1,), (0,)), ((), ())),
        preferred_element_type=jnp.float32)
    denom = red[:, 0:1]
    loc = red[:, 1:2]
    gate = 1.0 / denom

    me_part = jnp.sum(exps * gate, axis=0, keepdims=True)
    ce_part = jnp.sum(mask, axis=0, keepdims=True)

    idx_ref[...] = idx_f.astype(jnp.int32)
    loc_ref[...] = loc.astype(jnp.int32)
    gate_ref[...] = gate
    me_acc[...] += me_part
    cnt_acc[...] += ce_part

    @pl.when(i == NUM_BLOCKS - 1)
    def _fin():
        laux_ref[0, 0] = (jnp.sum(me_acc[...] * cnt_acc[...])
                          * (NUM_EXPERTS / (NUM_TOKENS * NUM_TOKENS)))


def kernel(input, W):
    num_tokens, num_experts = NUM_TOKENS, NUM_EXPERTS
    capacity = int((num_tokens + num_experts - 1) // num_experts)

    idx2, loc2, gate2, laux = pl.pallas_call(
        _gate_body,
        grid=(NUM_BLOCKS,),
        in_specs=[
            pl.BlockSpec((BLOCK_T, MODEL_DIM), lambda i: (i, 0)),
            pl.BlockSpec((NUM_EXPERTS, MODEL_DIM), lambda i: (0, 0)),
        ],
        out_specs=[
            pl.BlockSpec((BLOCK_T, 1), lambda i: (i, 0)),
            pl.BlockSpec((BLOCK_T, 1), lambda i: (i, 0)),
            pl.BlockSpec((BLOCK_T, 1), lambda i: (i, 0)),
            pl.BlockSpec(memory_space=pltpu.SMEM),
        ],
        out_shape=[
            jax.ShapeDtypeStruct((NUM_TOKENS, 1), jnp.int32),
            jax.ShapeDtypeStruct((NUM_TOKENS, 1), jnp.int32),
            jax.ShapeDtypeStruct((NUM_TOKENS, 1), jnp.float32),
            jax.ShapeDtypeStruct((1, 1), jnp.float32),
        ],
        scratch_shapes=[
            pltpu.VMEM((1, NUM_EXPERTS), jnp.float32),
            pltpu.VMEM((1, NUM_EXPERTS), jnp.float32),
        ],
    )(input, W)

    return (laux[0, 0], idx2[:, 0], capacity, loc2[:, 0], gate2[:, 0],
            num_experts)
